# baseline (speedup 1.0000x reference)
import numpy as np
import jax
import jax.numpy as jnp
from jax.experimental import pallas as pl
from jax.experimental.pallas import tpu as pltpu

PAD = 4
EPS = 1e-5
DYS = (-4, -2, -1, 0, 1, 2, 4)
LT = 128


def _band_table(w1, w31, w32, w33, W, C):
    n_dy = len(DYS)
    gidx = {dy: i for i, dy in enumerate(DYS)}
    nslot = 2 * PAD + 1

    m1 = w1[:, :, 0, 0].T[None]
    mk = [wk.transpose(2, 3, 1, 0).reshape(9, C, C) for wk in (w31, w32, w33)]
    m_all = jnp.concatenate([m1] + mk, axis=0)

    place = np.zeros((1 + 27, n_dy * nslot), np.float32)
    place[0, gidx[0] * nslot + PAD] = 1.0
    k = 1
    for d in (1, 2, 4):
        for ky in range(3):
            for kx in range(3):
                place[k, gidx[(ky - 1) * d] * nslot + (kx - 1) * d + PAD] = 1.0
                k += 1
    tab = jnp.einsum('kp,kab->pab', jnp.asarray(place), m_all)
    tab = tab.reshape(n_dy, nslot, C, C)

    tabr = tab[:, ::-1].transpose(0, 2, 1, 3).reshape(n_dy, C, nslot * C)
    lw = (W + 2 * PAD - 1 + W) * C
    lw = ((lw + 127) // 128) * 128
    base = (W - 1) * C
    wide = jnp.pad(tabr, ((0, 0), (0, 0), (base, lw - base - nslot * C)))
    return wide.astype(jnp.bfloat16)


def _make_conv_body(C, H, W):
    n_dy = len(DYS)
    WC = W * C

    def _conv_body(xp_ref, wide_ref, bias_ref, feat_ref, stat_ref, w_scr):
        TB = xp_ref.shape[0]
        nt = WC // LT

        @pl.when(pl.program_id(0) == 0)
        def _fold():
            for g in range(n_dy):
                for wi in range(W):
                    st = (W + PAD - 1 - wi) * C
                    w_scr[g, wi * C:(wi + 1) * C, :] = \
                        wide_ref[g, :, st:st + WC]

        xb = xp_ref[...]
        xs = []
        for dy in DYS:
            lo, hi = max(0, dy), min(H, H + dy)
            sl = xb[:, lo:hi, :]
            if dy < 0:
                sl = jnp.concatenate(
                    [jnp.zeros((TB, -dy, WC), jnp.bfloat16), sl], axis=1)
            elif dy > 0:
                sl = jnp.concatenate(
                    [sl, jnp.zeros((TB, dy, WC), jnp.bfloat16)], axis=1)
            xs.append(sl.reshape(TB * H, WC))
        cols = []
        for j in range(nt):
            k0, k1 = max(0, j - 1) * LT, min(nt, j + 2) * LT
            acc = jnp.zeros((TB * H, LT), jnp.float32)
            for i in range(n_dy):
                acc = acc + jnp.dot(xs[i][:, k0:k1],
                                    w_scr[i, k0:k1, j * LT:(j + 1) * LT],
                                    preferred_element_type=jnp.float32)
            cols.append(acc)
        feat = jnp.concatenate(cols, axis=1) + bias_ref[...]
        f3 = feat.reshape(TB, H, WC)
        feat_ref[...] = f3.astype(jnp.bfloat16)
        s = jnp.sum(f3, axis=1)
        sq = jnp.sum(f3 * f3, axis=1)
        stat_ref[...] = jnp.concatenate([s[:, None, :], sq[:, None, :]],
                                        axis=1)
    return _conv_body


def _make_tail_body(W, C, HW, B, TB2):
    HI = jax.lax.Precision.HIGHEST

    def _tail_body(feat_ref, stat_ref, rd_ref, rb_ref, wfc1t_ref, bfc1_ref,
                   wfc2t_ref, bfc2_ref, gamma_ref, beta_ref, out_ref,
                   ss_scr):
        b = pl.program_id(0)

        @pl.when(b == 0)
        def _glue():
            sum_c = jnp.dot(stat_ref[:, 0, :], rd_ref[...],
                            preferred_element_type=jnp.float32, precision=HI)
            sq_c = jnp.dot(stat_ref[:, 1, :], rd_ref[...],
                           preferred_element_type=jnp.float32, precision=HI)
            hid = jnp.maximum(
                jnp.dot(sum_c * (1.0 / HW), wfc1t_ref[...],
                        preferred_element_type=jnp.float32) + bfc1_ref[...],
                0.0)
            cw = jax.nn.sigmoid(
                jnp.dot(hid, wfc2t_ref[...],
                        preferred_element_type=jnp.float32) + bfc2_ref[...])
            tot = B * HW
            mu = jnp.sum(cw * sum_c, axis=0, keepdims=True) / tot
            ex2 = jnp.sum(cw * cw * sq_c, axis=0, keepdims=True) / tot
            var = jnp.maximum(ex2 - mu * mu, 0.0)
            inv = gamma_ref[...] * jax.lax.rsqrt(var + EPS)
            scale = cw * inv
            shift = beta_ref[...] - mu * inv
            ss_scr[:B, :] = jnp.dot(scale, rb_ref[...],
                                    preferred_element_type=jnp.float32,
                                    precision=HI)
            ss_scr[B:, :] = jnp.dot(shift, rb_ref[...],
                                    preferred_element_type=jnp.float32,
                                    precision=HI)

        sc = ss_scr[pl.ds(b * TB2, TB2), :]
        sh = ss_scr[B:, :]
        out_ref[...] = jnp.maximum(
            feat_ref[...].astype(jnp.float32) * sc[:, None, :]
            + sh[None, :, :], 0.0)
    return _tail_body


def kernel(x, w1, b1, w31, b31, w32, b32, w33, b33,
           wfc1, bfc1, wfc2, bfc2, gamma, beta):
    B, C, H, W = x.shape
    WC = W * C
    HW = H * W
    n_dy = len(DYS)

    wide = _band_table(w1, w31, w32, w33, W, C)
    bias_ld = jnp.tile(b1 + b31 + b32 + b33, W).reshape(1, WC)

    rd_np = np.zeros((WC, C), np.float32)
    rd_np[np.arange(WC), np.arange(WC) % C] = 1.0
    rd = jnp.asarray(rd_np)
    rb = jnp.asarray(rd_np.T)

    x_ld = jnp.transpose(x, (0, 2, 3, 1)).reshape(B, H, WC).astype(jnp.bfloat16)

    TB = 32
    nb = B // TB
    conv_cost = pl.CostEstimate(
        flops=2 * B * H * n_dy * WC * (WC * 5 // 8),
        transcendentals=0,
        bytes_accessed=4 * (x_ld.size + 2 * B * WC)
        + 2 * (B * H * WC + wide.size))

    feat, stats = pl.pallas_call(
        _make_conv_body(C, H, W),
        out_shape=(jax.ShapeDtypeStruct((B, H, WC), jnp.bfloat16),
                   jax.ShapeDtypeStruct((B, 2, WC), jnp.float32)),
        grid=(nb,),
        in_specs=[pl.BlockSpec((TB, H, WC), lambda b: (b, 0, 0)),
                  pl.BlockSpec(wide.shape, lambda b: (0, 0, 0)),
                  pl.BlockSpec((1, WC), lambda b: (0, 0))],
        out_specs=(pl.BlockSpec((TB, H, WC), lambda b: (b, 0, 0)),
                   pl.BlockSpec((TB, 2, WC), lambda b: (b, 0, 0))),
        scratch_shapes=[pltpu.VMEM((n_dy, WC, WC), jnp.bfloat16)],
        compiler_params=pltpu.CompilerParams(
            dimension_semantics=("arbitrary",)),
        cost_estimate=conv_cost,
    )(x_ld, wide, bias_ld)

    TB2 = 32
    tail_cost = pl.CostEstimate(
        flops=2 * B * H * WC, transcendentals=0,
        bytes_accessed=4 * (B * H * WC + 2 * B * WC) + 2 * B * H * WC)
    out_ld = pl.pallas_call(
        _make_tail_body(W, C, HW, B, TB2),
        out_shape=jax.ShapeDtypeStruct((B, H, WC), jnp.float32),
        grid=(B // TB2,),
        in_specs=[pl.BlockSpec((TB2, H, WC), lambda b: (b, 0, 0)),
                  pl.BlockSpec((B, 2, WC), lambda b: (0, 0, 0)),
                  pl.BlockSpec((WC, C), lambda b: (0, 0)),
                  pl.BlockSpec((C, WC), lambda b: (0, 0)),
                  pl.BlockSpec(wfc1.T.shape, lambda b: (0, 0)),
                  pl.BlockSpec((1, wfc1.shape[0]), lambda b: (0, 0)),
                  pl.BlockSpec(wfc2.T.shape, lambda b: (0, 0)),
                  pl.BlockSpec((1, C), lambda b: (0, 0)),
                  pl.BlockSpec((1, C), lambda b: (0, 0)),
                  pl.BlockSpec((1, C), lambda b: (0, 0))],
        out_specs=pl.BlockSpec((TB2, H, WC), lambda b: (b, 0, 0)),
        scratch_shapes=[pltpu.VMEM((B + 1, WC), jnp.float32)],
        compiler_params=pltpu.CompilerParams(
            dimension_semantics=("arbitrary",)),
        cost_estimate=tail_cost,
    )(feat, stats, rd, rb, wfc1.T, bfc1.reshape(1, -1), wfc2.T,
      bfc2.reshape(1, -1), gamma.reshape(1, -1), beta.reshape(1, -1))

    out_nhwc = out_ld.reshape(B, H, W, C)
    return jnp.transpose(out_nhwc, (0, 3, 1, 2))


# speedup vs baseline: 1.0106x; 1.0106x over previous
import numpy as np
import jax
import jax.numpy as jnp
from jax.experimental import pallas as pl
from jax.experimental.pallas import tpu as pltpu

PAD = 4
EPS = 1e-5
DYS = (-4, -2, -1, 0, 1, 2, 4)
LT = 128


def _band_table(w1, w31, w32, w33, W, C):
    n_dy = len(DYS)
    gidx = {dy: i for i, dy in enumerate(DYS)}
    nslot = 2 * PAD + 1

    m1 = w1[:, :, 0, 0].T[None]
    mk = [wk.transpose(2, 3, 1, 0).reshape(9, C, C) for wk in (w31, w32, w33)]
    m_all = jnp.concatenate([m1] + mk, axis=0)

    place = np.zeros((1 + 27, n_dy * nslot), np.float32)
    place[0, gidx[0] * nslot + PAD] = 1.0
    k = 1
    for d in (1, 2, 4):
        for ky in range(3):
            for kx in range(3):
                place[k, gidx[(ky - 1) * d] * nslot + (kx - 1) * d + PAD] = 1.0
                k += 1
    tab = jnp.einsum('kp,kab->pab', jnp.asarray(place), m_all)
    tab = tab.reshape(n_dy, nslot, C, C)

    tabr = tab[:, ::-1].transpose(0, 2, 1, 3).reshape(n_dy, C, nslot * C)
    lw = (W + 2 * PAD - 1 + W) * C
    lw = ((lw + 127) // 128) * 128
    base = (W - 1) * C
    wide = jnp.pad(tabr, ((0, 0), (0, 0), (base, lw - base - nslot * C)))
    return wide.astype(jnp.bfloat16)


def _make_conv_body(C, H, W):
    n_dy = len(DYS)
    WC = W * C

    def _conv_body(xp_ref, wide_ref, bias_ref, feat_ref, stat_ref, w_scr):
        TB = xp_ref.shape[0]
        nt = WC // LT

        @pl.when(pl.program_id(0) == 0)
        def _fold():
            for g in range(n_dy):
                for wi in range(W):
                    st = (W + PAD - 1 - wi) * C
                    w_scr[g, wi * C:(wi + 1) * C, :] = \
                        wide_ref[g, :, st:st + WC]

        xb = xp_ref[...].astype(jnp.bfloat16)
        xs = []
        for dy in DYS:
            lo, hi = max(0, dy), min(H, H + dy)
            sl = xb[:, lo:hi, :]
            if dy < 0:
                sl = jnp.concatenate(
                    [jnp.zeros((TB, -dy, WC), jnp.bfloat16), sl], axis=1)
            elif dy > 0:
                sl = jnp.concatenate(
                    [sl, jnp.zeros((TB, dy, WC), jnp.bfloat16)], axis=1)
            xs.append(sl.reshape(TB * H, WC))
        cols = []
        for j in range(nt):
            k0, k1 = max(0, j - 1) * LT, min(nt, j + 2) * LT
            acc = jnp.zeros((TB * H, LT), jnp.float32)
            for i in range(n_dy):
                acc = acc + jnp.dot(xs[i][:, k0:k1],
                                    w_scr[i, k0:k1, j * LT:(j + 1) * LT],
                                    preferred_element_type=jnp.float32)
            cols.append(acc)
        feat = jnp.concatenate(cols, axis=1) + bias_ref[...]
        f3 = feat.reshape(TB, H, WC)
        feat_ref[...] = f3.astype(jnp.bfloat16)
        s = jnp.sum(f3, axis=1)
        sq = jnp.sum(f3 * f3, axis=1)
        stat_ref[...] = jnp.concatenate([s[:, None, :], sq[:, None, :]],
                                        axis=1)
    return _conv_body


def _make_tail_body(W, C, HW, B, TB2):
    HI = jax.lax.Precision.HIGHEST

    def _tail_body(feat_ref, stat_ref, rd_ref, rb_ref, wfc1t_ref, bfc1_ref,
                   wfc2t_ref, bfc2_ref, gamma_ref, beta_ref, out_ref,
                   ss_scr):
        b = pl.program_id(0)

        @pl.when(b == 0)
        def _glue():
            sum_c = jnp.dot(stat_ref[:, 0, :], rd_ref[...],
                            preferred_element_type=jnp.float32, precision=HI)
            sq_c = jnp.dot(stat_ref[:, 1, :], rd_ref[...],
                           preferred_element_type=jnp.float32, precision=HI)
            hid = jnp.maximum(
                jnp.dot(sum_c * (1.0 / HW), wfc1t_ref[...],
                        preferred_element_type=jnp.float32) + bfc1_ref[...],
                0.0)
            cw = jax.nn.sigmoid(
                jnp.dot(hid, wfc2t_ref[...],
                        preferred_element_type=jnp.float32) + bfc2_ref[...])
            tot = B * HW
            mu = jnp.sum(cw * sum_c, axis=0, keepdims=True) / tot
            ex2 = jnp.sum(cw * cw * sq_c, axis=0, keepdims=True) / tot
            var = jnp.maximum(ex2 - mu * mu, 0.0)
            inv = gamma_ref[...] * jax.lax.rsqrt(var + EPS)
            scale = cw * inv
            shift = beta_ref[...] - mu * inv
            ss_scr[:B, :] = jnp.dot(scale, rb_ref[...],
                                    preferred_element_type=jnp.float32,
                                    precision=HI)
            ss_scr[B:, :] = jnp.dot(shift, rb_ref[...],
                                    preferred_element_type=jnp.float32,
                                    precision=HI)

        sc = ss_scr[pl.ds(b * TB2, TB2), :]
        sh = ss_scr[B:, :]
        out_ref[...] = jnp.maximum(
            feat_ref[...].astype(jnp.float32) * sc[:, None, :]
            + sh[None, :, :], 0.0)
    return _tail_body


def kernel(x, w1, b1, w31, b31, w32, b32, w33, b33,
           wfc1, bfc1, wfc2, bfc2, gamma, beta):
    B, C, H, W = x.shape
    WC = W * C
    HW = H * W
    n_dy = len(DYS)

    wide = _band_table(w1, w31, w32, w33, W, C)
    bias_ld = jnp.tile(b1 + b31 + b32 + b33, W).reshape(1, WC)

    LP = LT
    rd_np = np.zeros((WC, LP), np.float32)
    rd_np[np.arange(WC), np.arange(WC) % C] = 1.0
    rd = jnp.asarray(rd_np)
    rb = jnp.asarray(rd_np.T)
    CH4 = wfc1.shape[0]
    wfc1t_p = jnp.pad(wfc1.T, ((0, LP - C), (0, LP - CH4)))
    bfc1_p = jnp.pad(bfc1.reshape(1, -1), ((0, 0), (0, LP - CH4)))
    wfc2t_p = jnp.pad(wfc2.T, ((0, LP - CH4), (0, LP - C)))
    bfc2_p = jnp.pad(bfc2.reshape(1, -1), ((0, 0), (0, LP - C)))
    gamma_p = jnp.pad(gamma.reshape(1, -1), ((0, 0), (0, LP - C)))
    beta_p = jnp.pad(beta.reshape(1, -1), ((0, 0), (0, LP - C)))

    x_ld = jnp.transpose(x, (0, 2, 3, 1)).reshape(B, H, WC)

    TB = 32
    nb = B // TB
    conv_cost = pl.CostEstimate(
        flops=2 * B * H * n_dy * WC * (WC * 5 // 8),
        transcendentals=0,
        bytes_accessed=4 * (x_ld.size + 2 * B * WC)
        + 2 * (B * H * WC + wide.size))

    feat, stats = pl.pallas_call(
        _make_conv_body(C, H, W),
        out_shape=(jax.ShapeDtypeStruct((B, H, WC), jnp.bfloat16),
                   jax.ShapeDtypeStruct((B, 2, WC), jnp.float32)),
        grid=(nb,),
        in_specs=[pl.BlockSpec((TB, H, WC), lambda b: (b, 0, 0)),
                  pl.BlockSpec(wide.shape, lambda b: (0, 0, 0)),
                  pl.BlockSpec((1, WC), lambda b: (0, 0))],
        out_specs=(pl.BlockSpec((TB, H, WC), lambda b: (b, 0, 0)),
                   pl.BlockSpec((TB, 2, WC), lambda b: (b, 0, 0))),
        scratch_shapes=[pltpu.VMEM((n_dy, WC, WC), jnp.bfloat16)],
        compiler_params=pltpu.CompilerParams(
            dimension_semantics=("arbitrary",)),
        cost_estimate=conv_cost,
    )(x_ld, wide, bias_ld)

    TB2 = 32
    tail_cost = pl.CostEstimate(
        flops=2 * B * H * WC, transcendentals=0,
        bytes_accessed=4 * (B * H * WC + 2 * B * WC) + 2 * B * H * WC)
    out_ld = pl.pallas_call(
        _make_tail_body(W, C, HW, B, TB2),
        out_shape=jax.ShapeDtypeStruct((B, H, WC), jnp.float32),
        grid=(B // TB2,),
        in_specs=[pl.BlockSpec((TB2, H, WC), lambda b: (b, 0, 0)),
                  pl.BlockSpec((B, 2, WC), lambda b: (0, 0, 0)),
                  pl.BlockSpec((WC, LP), lambda b: (0, 0)),
                  pl.BlockSpec((LP, WC), lambda b: (0, 0)),
                  pl.BlockSpec((LP, LP), lambda b: (0, 0)),
                  pl.BlockSpec((1, LP), lambda b: (0, 0)),
                  pl.BlockSpec((LP, LP), lambda b: (0, 0)),
                  pl.BlockSpec((1, LP), lambda b: (0, 0)),
                  pl.BlockSpec((1, LP), lambda b: (0, 0)),
                  pl.BlockSpec((1, LP), lambda b: (0, 0))],
        out_specs=pl.BlockSpec((TB2, H, WC), lambda b: (b, 0, 0)),
        scratch_shapes=[pltpu.VMEM((B + 1, WC), jnp.float32)],
        compiler_params=pltpu.CompilerParams(
            dimension_semantics=("arbitrary",)),
        cost_estimate=tail_cost,
    )(feat, stats, rd, rb, wfc1t_p, bfc1_p, wfc2t_p,
      bfc2_p, gamma_p, beta_p)

    out_nhwc = out_ld.reshape(B, H, W, C)
    return jnp.transpose(out_nhwc, (0, 3, 1, 2))


# speedup vs baseline: 1.0133x; 1.0026x over previous
import numpy as np
import jax
import jax.numpy as jnp
from jax.experimental import pallas as pl
from jax.experimental.pallas import tpu as pltpu

PAD = 4
EPS = 1e-5
DYS = (-4, -2, -1, 0, 1, 2, 4)
LT = 128


def _band_table(w1, w31, w32, w33, W, C):
    n_dy = len(DYS)
    gidx = {dy: i for i, dy in enumerate(DYS)}
    nslot = 2 * PAD + 1

    m1 = w1[:, :, 0, 0].T[None]
    mk = [wk.transpose(2, 3, 1, 0).reshape(9, C, C) for wk in (w31, w32, w33)]
    m_all = jnp.concatenate([m1] + mk, axis=0)

    place = np.zeros((1 + 27, n_dy * nslot), np.float32)
    place[0, gidx[0] * nslot + PAD] = 1.0
    k = 1
    for d in (1, 2, 4):
        for ky in range(3):
            for kx in range(3):
                place[k, gidx[(ky - 1) * d] * nslot + (kx - 1) * d + PAD] = 1.0
                k += 1
    tab = jnp.einsum('kp,kab->pab', jnp.asarray(place), m_all)
    tab = tab.reshape(n_dy, nslot, C, C)

    tabr = tab[:, ::-1].transpose(0, 2, 1, 3).reshape(n_dy, C, nslot * C)
    lw = (W + 2 * PAD - 1 + W) * C
    lw = ((lw + 127) // 128) * 128
    base = (W - 1) * C
    wide = jnp.pad(tabr, ((0, 0), (0, 0), (base, lw - base - nslot * C)))
    return wide.astype(jnp.bfloat16)


def _make_conv_body(C, H, W):
    n_dy = len(DYS)
    WC = W * C

    def _conv_body(xp_ref, wide_ref, bias_ref, feat_ref, stat_ref, w_scr):
        TB = xp_ref.shape[0]
        nt = WC // LT

        @pl.when(pl.program_id(0) == 0)
        def _fold():
            for g in range(n_dy):
                for wi in range(W):
                    st = (W + PAD - 1 - wi) * C
                    w_scr[g, wi * C:(wi + 1) * C, :] = \
                        wide_ref[g, :, st:st + WC]

        xb = xp_ref[...].astype(jnp.bfloat16)
        xs = []
        for dy in DYS:
            lo, hi = max(0, dy), min(H, H + dy)
            sl = xb[:, lo:hi, :]
            if dy < 0:
                sl = jnp.concatenate(
                    [jnp.zeros((TB, -dy, WC), jnp.bfloat16), sl], axis=1)
            elif dy > 0:
                sl = jnp.concatenate(
                    [sl, jnp.zeros((TB, dy, WC), jnp.bfloat16)], axis=1)
            xs.append(sl.reshape(TB * H, WC))
        cols = []
        for j in range(nt):
            k0, k1 = max(0, j - 1) * LT, min(nt, j + 2) * LT
            acc = jnp.zeros((TB * H, LT), jnp.float32)
            for i in range(n_dy):
                acc = acc + jnp.dot(xs[i][:, k0:k1],
                                    w_scr[i, k0:k1, j * LT:(j + 1) * LT],
                                    preferred_element_type=jnp.float32)
            cols.append(acc)
        feat = jnp.concatenate(cols, axis=1) + bias_ref[...]
        f3 = feat.reshape(TB, H, WC)
        feat_ref[...] = f3.astype(jnp.bfloat16)
        s = jnp.sum(f3, axis=1)
        sq = jnp.sum(f3 * f3, axis=1)
        stat_ref[...] = jnp.concatenate([s[:, None, :], sq[:, None, :]],
                                        axis=1)
    return _conv_body


def _make_tail_body(W, C, HW, B, TB2):
    HI = jax.lax.Precision.HIGHEST

    def _tail_body(feat_ref, stat_ref, rd_ref, rb_ref, wfc1t_ref, bfc1_ref,
                   wfc2t_ref, bfc2_ref, gamma_ref, beta_ref, out_ref,
                   ss_scr):
        b = pl.program_id(0)

        @pl.when(b == 0)
        def _glue():
            sum_c = jnp.dot(stat_ref[:, 0, :], rd_ref[...],
                            preferred_element_type=jnp.float32, precision=HI)
            sq_c = jnp.dot(stat_ref[:, 1, :], rd_ref[...],
                           preferred_element_type=jnp.float32, precision=HI)
            hid = jnp.maximum(
                jnp.dot(sum_c * (1.0 / HW), wfc1t_ref[...],
                        preferred_element_type=jnp.float32) + bfc1_ref[...],
                0.0)
            cw = jax.nn.sigmoid(
                jnp.dot(hid, wfc2t_ref[...],
                        preferred_element_type=jnp.float32) + bfc2_ref[...])
            tot = B * HW
            mu = jnp.sum(cw * sum_c, axis=0, keepdims=True) / tot
            ex2 = jnp.sum(cw * cw * sq_c, axis=0, keepdims=True) / tot
            var = jnp.maximum(ex2 - mu * mu, 0.0)
            inv = gamma_ref[...] * jax.lax.rsqrt(var + EPS)
            scale = cw * inv
            shift = beta_ref[...] - mu * inv
            ss_scr[:B, :] = jnp.dot(scale, rb_ref[...],
                                    preferred_element_type=jnp.float32,
                                    precision=HI)
            ss_scr[B:, :] = jnp.dot(shift, rb_ref[...],
                                    preferred_element_type=jnp.float32,
                                    precision=HI)

        sc = ss_scr[pl.ds(b * TB2, TB2), :]
        sh = ss_scr[B:, :]
        out_ref[...] = jnp.maximum(
            feat_ref[...].astype(jnp.float32) * sc[:, None, :]
            + sh[None, :, :], 0.0)
    return _tail_body


def kernel(x, w1, b1, w31, b31, w32, b32, w33, b33,
           wfc1, bfc1, wfc2, bfc2, gamma, beta):
    B, C, H, W = x.shape
    WC = W * C
    HW = H * W
    n_dy = len(DYS)

    wide = _band_table(w1, w31, w32, w33, W, C)
    bias_ld = jnp.tile(b1 + b31 + b32 + b33, W).reshape(1, WC)

    LP = LT
    rd_np = np.zeros((WC, LP), np.float32)
    rd_np[np.arange(WC), np.arange(WC) % C] = 1.0
    rd = jnp.asarray(rd_np)
    rb = jnp.asarray(rd_np.T)
    CH4 = wfc1.shape[0]
    wfc1t_p = jnp.pad(wfc1.T, ((0, LP - C), (0, LP - CH4)))
    bfc1_p = jnp.pad(bfc1.reshape(1, -1), ((0, 0), (0, LP - CH4)))
    wfc2t_p = jnp.pad(wfc2.T, ((0, LP - CH4), (0, LP - C)))
    bfc2_p = jnp.pad(bfc2.reshape(1, -1), ((0, 0), (0, LP - C)))
    gamma_p = jnp.pad(gamma.reshape(1, -1), ((0, 0), (0, LP - C)))
    beta_p = jnp.pad(beta.reshape(1, -1), ((0, 0), (0, LP - C)))

    x_ld = jnp.transpose(x, (0, 2, 3, 1)).reshape(B, H, WC)

    TB = 64
    nb = B // TB
    conv_cost = pl.CostEstimate(
        flops=2 * B * H * n_dy * WC * (WC * 5 // 8),
        transcendentals=0,
        bytes_accessed=4 * (x_ld.size + 2 * B * WC)
        + 2 * (B * H * WC + wide.size))

    feat, stats = pl.pallas_call(
        _make_conv_body(C, H, W),
        out_shape=(jax.ShapeDtypeStruct((B, H, WC), jnp.bfloat16),
                   jax.ShapeDtypeStruct((B, 2, WC), jnp.float32)),
        grid=(nb,),
        in_specs=[pl.BlockSpec((TB, H, WC), lambda b: (b, 0, 0)),
                  pl.BlockSpec(wide.shape, lambda b: (0, 0, 0)),
                  pl.BlockSpec((1, WC), lambda b: (0, 0))],
        out_specs=(pl.BlockSpec((TB, H, WC), lambda b: (b, 0, 0)),
                   pl.BlockSpec((TB, 2, WC), lambda b: (b, 0, 0))),
        scratch_shapes=[pltpu.VMEM((n_dy, WC, WC), jnp.bfloat16)],
        compiler_params=pltpu.CompilerParams(
            dimension_semantics=("arbitrary",)),
        cost_estimate=conv_cost,
    )(x_ld, wide, bias_ld)

    TB2 = 32
    tail_cost = pl.CostEstimate(
        flops=2 * B * H * WC, transcendentals=0,
        bytes_accessed=4 * (B * H * WC + 2 * B * WC) + 2 * B * H * WC)
    out_ld = pl.pallas_call(
        _make_tail_body(W, C, HW, B, TB2),
        out_shape=jax.ShapeDtypeStruct((B, H, WC), jnp.float32),
        grid=(B // TB2,),
        in_specs=[pl.BlockSpec((TB2, H, WC), lambda b: (b, 0, 0)),
                  pl.BlockSpec((B, 2, WC), lambda b: (0, 0, 0)),
                  pl.BlockSpec((WC, LP), lambda b: (0, 0)),
                  pl.BlockSpec((LP, WC), lambda b: (0, 0)),
                  pl.BlockSpec((LP, LP), lambda b: (0, 0)),
                  pl.BlockSpec((1, LP), lambda b: (0, 0)),
                  pl.BlockSpec((LP, LP), lambda b: (0, 0)),
                  pl.BlockSpec((1, LP), lambda b: (0, 0)),
                  pl.BlockSpec((1, LP), lambda b: (0, 0)),
                  pl.BlockSpec((1, LP), lambda b: (0, 0))],
        out_specs=pl.BlockSpec((TB2, H, WC), lambda b: (b, 0, 0)),
        scratch_shapes=[pltpu.VMEM((B + 1, WC), jnp.float32)],
        compiler_params=pltpu.CompilerParams(
            dimension_semantics=("arbitrary",)),
        cost_estimate=tail_cost,
    )(feat, stats, rd, rb, wfc1t_p, bfc1_p, wfc2t_p,
      bfc2_p, gamma_p, beta_p)

    out_nhwc = out_ld.reshape(B, H, W, C)
    return jnp.transpose(out_nhwc, (0, 3, 1, 2))
